# baseline (speedup 1.0000x reference)
import functools

import jax
import jax.numpy as jnp
from jax import lax
from jax.experimental import pallas as pl
from jax.experimental.pallas import tpu as pltpu
from jax.experimental.pallas import tpu_sc as plsc

_TC_GRID = 16

_D = 768
_VECS_PER_ROW = _D // 16
_CHUNK_ROWS = 64
_CHUNK_ELEMS = _CHUNK_ROWS * _D


def _tc_add_kernel(x_ref, tab_ref, o_ref):
    o_ref[...] = x_ref[...] + tab_ref[0:1, :]


def _tc_broadcast_add(x2d, row2d):
    n, d = x2d.shape
    blk = n // _TC_GRID
    return pl.pallas_call(
        _tc_add_kernel,
        grid=(_TC_GRID,),
        in_specs=[
            pl.BlockSpec((blk, d), lambda i: (i, 0)),
            pl.BlockSpec((1, d), lambda i: (0, 0)),
        ],
        out_specs=pl.BlockSpec((blk, d), lambda i: (i, 0)),
        out_shape=jax.ShapeDtypeStruct((n, d), x2d.dtype),
    )(x2d, row2d)


def _sc_broadcast_add(xflat, row):
    n_elems = xflat.shape[0]
    info = plsc.get_sparse_core_info()
    nw = info.num_cores * info.num_subcores
    elems_per_w = n_elems // nw
    chunks_per_w = elems_per_w // _CHUNK_ELEMS

    @functools.partial(
        pl.kernel,
        mesh=plsc.VectorSubcoreMesh(core_axis_name="c", subcore_axis_name="s"),
        out_type=jax.ShapeDtypeStruct((n_elems,), jnp.float32),
        scratch_types=[
            pltpu.VMEM((_CHUNK_ELEMS,), jnp.float32),
            pltpu.VMEM((_D,), jnp.float32),
        ],
    )
    def sc_kernel(x_hbm, row_hbm, out_hbm, buf, rowbuf):
        wid = lax.axis_index("s") * info.num_cores + lax.axis_index("c")
        base = wid * elems_per_w
        pltpu.sync_copy(row_hbm, rowbuf)

        def chunk_body(c, _):
            start = base + c * _CHUNK_ELEMS
            start = pl.multiple_of(start, _D)
            pltpu.sync_copy(x_hbm.at[pl.ds(start, _CHUNK_ELEMS)], buf)

            def row_body(r, _):
                off = pl.multiple_of(r * _D, _D)
                for j in range(_VECS_PER_ROW):
                    sl = pl.ds(off + j * 16, 16)
                    buf[sl] = buf[sl] + rowbuf[pl.ds(j * 16, 16)]
                return 0

            lax.fori_loop(0, _CHUNK_ROWS, row_body, 0, unroll=False)
            pltpu.sync_copy(buf, out_hbm.at[pl.ds(start, _CHUNK_ELEMS)])
            return 0

        lax.fori_loop(0, chunks_per_w, chunk_body, 0, unroll=False)

    return sc_kernel(xflat, row)


def kernel(image_embeddings, text_embeddings, modal_table):
    b, li, d = image_embeddings.shape
    lt = text_embeddings.shape[1]
    img = _tc_broadcast_add(
        image_embeddings.reshape(b * li, d), modal_table[0:1, :]
    )
    txt = _sc_broadcast_add(
        text_embeddings.reshape(b * lt * d), modal_table[1]
    )
    return img.reshape(b, li, d), txt.reshape(b, lt, d)


# speedup vs baseline: 1.4726x; 1.4726x over previous
import functools

import jax
import jax.numpy as jnp
from jax import lax
from jax.experimental import pallas as pl
from jax.experimental.pallas import tpu as pltpu
from jax.experimental.pallas import tpu_sc as plsc

_TC_GRID = 16

_D = 768
_VECS_PER_ROW = _D // 16
_CHUNK_ROWS = 64
_CHUNK_ELEMS = _CHUNK_ROWS * _D


def _tc_add_kernel(x_ref, tab_ref, o_ref):
    o_ref[...] = x_ref[...] + tab_ref[0:1, :]


def _tc_broadcast_add(x2d, row2d):
    n, d = x2d.shape
    blk = n // _TC_GRID
    return pl.pallas_call(
        _tc_add_kernel,
        grid=(_TC_GRID,),
        in_specs=[
            pl.BlockSpec((blk, d), lambda i: (i, 0)),
            pl.BlockSpec((1, d), lambda i: (0, 0)),
        ],
        out_specs=pl.BlockSpec((blk, d), lambda i: (i, 0)),
        out_shape=jax.ShapeDtypeStruct((n, d), x2d.dtype),
    )(x2d, row2d)


def _sc_broadcast_add(xflat, row):
    n_elems = xflat.shape[0]
    info = plsc.get_sparse_core_info()
    nw = info.num_cores * info.num_subcores
    elems_per_w = n_elems // nw
    chunks_per_w = elems_per_w // _CHUNK_ELEMS

    @functools.partial(
        pl.kernel,
        mesh=plsc.VectorSubcoreMesh(core_axis_name="c", subcore_axis_name="s"),
        out_type=jax.ShapeDtypeStruct((n_elems,), jnp.float32),
        scratch_types=[
            pltpu.VMEM((_CHUNK_ELEMS,), jnp.float32),
            pltpu.VMEM((_CHUNK_ELEMS,), jnp.float32),
            pltpu.VMEM((_D,), jnp.float32),
            pltpu.SemaphoreType.DMA,
            pltpu.SemaphoreType.DMA,
            pltpu.SemaphoreType.DMA,
            pltpu.SemaphoreType.DMA,
        ],
    )
    def sc_kernel(x_hbm, row_hbm, out_hbm, buf0, buf1, rowbuf,
                  in_sem0, in_sem1, out_sem0, out_sem1):
        wid = lax.axis_index("s") * info.num_cores + lax.axis_index("c")
        base = wid * elems_per_w
        bufs = (buf0, buf1)
        in_sems = (in_sem0, in_sem1)
        out_sems = (out_sem0, out_sem1)
        pltpu.sync_copy(row_hbm, rowbuf)
        rvs = [rowbuf[pl.ds(j * 16, 16)] for j in range(_VECS_PER_ROW)]

        def hbm_slice(c):
            start = pl.multiple_of(base + c * _CHUNK_ELEMS, _D)
            return pl.ds(start, _CHUNK_ELEMS)

        def in_copy(c):
            b = c % 2
            return pltpu.make_async_copy(
                x_hbm.at[hbm_slice(c)], bufs[b], in_sems[b])

        def out_copy(c):
            b = c % 2
            return pltpu.make_async_copy(
                bufs[b], out_hbm.at[hbm_slice(c)], out_sems[b])

        in_copy(0).start()
        for c in range(chunks_per_w):
            b = c % 2
            if c + 1 < chunks_per_w:
                if c >= 1:
                    out_copy(c - 1).wait()
                in_copy(c + 1).start()
            buf = bufs[b]

            def row_body(r, _):
                off = pl.multiple_of(r * _D, _D)
                for j in range(_VECS_PER_ROW):
                    sl = pl.ds(off + j * 16, 16)
                    buf[sl] = buf[sl] + rvs[j]
                return 0

            in_copy(c).wait()
            lax.fori_loop(0, _CHUNK_ROWS, row_body, 0, unroll=False)
            out_copy(c).start()
        if chunks_per_w >= 2:
            out_copy(chunks_per_w - 2).wait()
        out_copy(chunks_per_w - 1).wait()

    return sc_kernel(xflat, row)


def kernel(image_embeddings, text_embeddings, modal_table):
    b, li, d = image_embeddings.shape
    lt = text_embeddings.shape[1]
    img = _tc_broadcast_add(
        image_embeddings.reshape(b * li, d), modal_table[0:1, :]
    )
    txt = _sc_broadcast_add(
        text_embeddings.reshape(b * lt * d), modal_table[1]
    )
    return img.reshape(b, li, d), txt.reshape(b, lt, d)


# speedup vs baseline: 4.0088x; 2.7223x over previous
import jax
import jax.numpy as jnp
from jax.experimental import pallas as pl

_GRID = 16


def _add_rows_kernel(img_ref, txt_ref, tab_ref, img_out_ref, txt_out_ref):
    img_out_ref[...] = img_ref[...] + tab_ref[0:1, :]
    txt_out_ref[...] = txt_ref[...] + tab_ref[1:2, :]


def kernel(image_embeddings, text_embeddings, modal_table):
    b, li, d = image_embeddings.shape
    lt = text_embeddings.shape[1]
    ni, nt = b * li, b * lt
    bi, bt = ni // _GRID, nt // _GRID
    img2d = image_embeddings.reshape(ni, d)
    txt2d = text_embeddings.reshape(nt, d)
    img, txt = pl.pallas_call(
        _add_rows_kernel,
        grid=(_GRID,),
        in_specs=[
            pl.BlockSpec((bi, d), lambda i: (i, 0)),
            pl.BlockSpec((bt, d), lambda i: (i, 0)),
            pl.BlockSpec((2, d), lambda i: (0, 0)),
        ],
        out_specs=[
            pl.BlockSpec((bi, d), lambda i: (i, 0)),
            pl.BlockSpec((bt, d), lambda i: (i, 0)),
        ],
        out_shape=[
            jax.ShapeDtypeStruct((ni, d), img2d.dtype),
            jax.ShapeDtypeStruct((nt, d), txt2d.dtype),
        ],
    )(img2d, txt2d, modal_table)
    return img.reshape(b, li, d), txt.reshape(b, lt, d)
